# Initial kernel scaffold; baseline (speedup 1.0000x reference)
import jax
import jax.numpy as jnp
from jax.experimental import pallas as pl


def kernel(x, table, gamma, beta, W_ih, W_hh, b_ih, b_hh, W_cls, b_cls):
    raise NotImplementedError("write your pallas kernel here")


# speedup vs baseline: 4.6133x; 4.6133x over previous
import functools

import jax
import jax.numpy as jnp
from jax import lax
from jax.experimental import pallas as pl
from jax.experimental.pallas import tpu as pltpu
from jax.experimental.pallas import tpu_sc as plsc

_NUM_CORES = 2
_NUM_SUBCORES = 16
_NUM_WORKERS = _NUM_CORES * _NUM_SUBCORES
_CHUNK = 128
_LANE = 128


def _make_ln_table_body(real_d):
    def _ln_table_body(table_ref, gamma_ref, beta_ref, out_ref):
        t = table_ref[...]
        lane = lax.broadcasted_iota(jnp.int32, t.shape, 1)
        mu = jnp.sum(t, axis=1, keepdims=True) * (1.0 / real_d)
        d = jnp.where(lane < real_d, t - mu, 0.0)
        var = jnp.sum(d * d, axis=1, keepdims=True) * (1.0 / real_d)
        out_ref[...] = d * lax.rsqrt(var + 1e-5) * gamma_ref[...] + beta_ref[...]
    return _ln_table_body


def _sc_gather(norm_table, idx_flat):
    n, d = idx_flat.shape[0], norm_table.shape[1]
    per_w = n // _NUM_WORKERS
    mesh = plsc.VectorSubcoreMesh(
        core_axis_name="c", subcore_axis_name="s",
        num_cores=_NUM_CORES, num_subcores=_NUM_SUBCORES)

    @functools.partial(
        pl.kernel,
        mesh=mesh,
        out_type=jax.ShapeDtypeStruct((n, d), jnp.float32),
        scratch_types=[
            pltpu.VMEM((per_w,), jnp.int32),
            pltpu.VMEM((_CHUNK, d), jnp.float32),
            pltpu.SemaphoreType.DMA,
        ],
    )
    def gather_k(tbl_hbm, idx_hbm, out_hbm, idx_v, rows_v, sem):
        wid = lax.axis_index("s") * _NUM_CORES + lax.axis_index("c")
        base = wid * per_w
        pltpu.sync_copy(idx_hbm.at[pl.ds(base, per_w)], idx_v)

        def chunk(j, carry):
            off = j * _CHUNK
            pltpu.async_copy(
                tbl_hbm.at[idx_v.at[pl.ds(off, _CHUNK)]], rows_v, sem).wait()
            pltpu.sync_copy(rows_v, out_hbm.at[pl.ds(base + off, _CHUNK)])
            return carry

        lax.fori_loop(0, per_w // _CHUNK, chunk, 0)

    return gather_k(norm_table, idx_flat)


def _lstm_body(emb_ref, wcat_ref, bias_ref, wcls_ref, bcls_ref, out_ref,
               h_scr, c_scr):
    t = pl.program_id(0)
    nt = pl.num_programs(0)
    h4 = bias_ref.shape[1]
    hdim = h4 // 4

    @pl.when(t == 0)
    def _():
        h_scr[...] = jnp.zeros_like(h_scr)
        c_scr[...] = jnp.zeros_like(c_scr)

    xt = emb_ref[0]
    h = h_scr[...]
    cat = jnp.concatenate([xt, h], axis=1)
    gates = jnp.dot(cat, wcat_ref[...],
                    preferred_element_type=jnp.float32) + bias_ref[...]
    i = jax.nn.sigmoid(gates[:, 0 * hdim:1 * hdim])
    f = jax.nn.sigmoid(gates[:, 1 * hdim:2 * hdim])
    g = jnp.tanh(gates[:, 2 * hdim:3 * hdim])
    o = jax.nn.sigmoid(gates[:, 3 * hdim:4 * hdim])
    c = f * c_scr[...] + i * g
    hn = o * jnp.tanh(c)
    h_scr[...] = hn
    c_scr[...] = c

    @pl.when(t == nt - 1)
    def _():
        out_ref[...] = jnp.dot(hn, wcls_ref[...],
                               preferred_element_type=jnp.float32) + bcls_ref[...]


def kernel(x, table, gamma, beta, W_ih, W_hh, b_ih, b_hh, W_cls, b_cls):
    B, T = x.shape
    V, D = table.shape
    H = W_hh.shape[1]
    C = W_cls.shape[0]
    pad = _LANE - D

    tbl128 = jnp.pad(table, ((0, 0), (0, pad)))
    gamma128 = jnp.pad(gamma.reshape(1, D), ((0, 0), (0, pad)))
    beta128 = jnp.pad(beta.reshape(1, D), ((0, 0), (0, pad)))

    norm_table = pl.pallas_call(
        _make_ln_table_body(D),
        out_shape=jax.ShapeDtypeStruct((V, _LANE), jnp.float32),
    )(tbl128, gamma128, beta128)

    idx_flat = x.T.reshape(-1)
    emb = _sc_gather(norm_table, idx_flat).reshape(T, B, _LANE)

    wcat = jnp.concatenate(
        [W_ih.T, jnp.zeros((pad, 4 * H), jnp.float32), W_hh.T], axis=0)
    bias = (b_ih + b_hh).reshape(1, 4 * H)

    return pl.pallas_call(
        _lstm_body,
        grid=(T,),
        in_specs=[
            pl.BlockSpec((1, B, _LANE), lambda t: (t, 0, 0)),
            pl.BlockSpec((_LANE + H, 4 * H), lambda t: (0, 0)),
            pl.BlockSpec((1, 4 * H), lambda t: (0, 0)),
            pl.BlockSpec((H, C), lambda t: (0, 0)),
            pl.BlockSpec((1, C), lambda t: (0, 0)),
        ],
        out_specs=pl.BlockSpec((B, C), lambda t: (0, 0)),
        out_shape=jax.ShapeDtypeStruct((B, C), jnp.float32),
        scratch_shapes=[
            pltpu.VMEM((B, H), jnp.float32),
            pltpu.VMEM((B, H), jnp.float32),
        ],
    )(emb, wcat, bias, W_cls.T, b_cls.reshape(1, C))


# speedup vs baseline: 5.2459x; 1.1371x over previous
import functools

import jax
import jax.numpy as jnp
from jax import lax
from jax.experimental import pallas as pl
from jax.experimental.pallas import tpu as pltpu
from jax.experimental.pallas import tpu_sc as plsc

_NUM_CORES = 2
_NUM_SUBCORES = 16
_NUM_WORKERS = _NUM_CORES * _NUM_SUBCORES
_CHUNK = 128
_LANE = 128


def _make_ln_table_body(real_d):
    def _ln_table_body(table_ref, gamma_ref, beta_ref, out_ref):
        t = table_ref[...]
        lane = lax.broadcasted_iota(jnp.int32, t.shape, 1)
        mu = jnp.sum(t, axis=1, keepdims=True) * (1.0 / real_d)
        d = jnp.where(lane < real_d, t - mu, 0.0)
        var = jnp.sum(d * d, axis=1, keepdims=True) * (1.0 / real_d)
        out_ref[...] = d * lax.rsqrt(var + 1e-5) * gamma_ref[...] + beta_ref[...]
    return _ln_table_body


def _sc_gather(norm_table, idx_flat):
    n, d = idx_flat.shape[0], norm_table.shape[1]
    per_w = n // _NUM_WORKERS
    mesh = plsc.VectorSubcoreMesh(
        core_axis_name="c", subcore_axis_name="s",
        num_cores=_NUM_CORES, num_subcores=_NUM_SUBCORES)

    @functools.partial(
        pl.kernel,
        mesh=mesh,
        out_type=jax.ShapeDtypeStruct((n, d), jnp.float32),
        scratch_types=[
            pltpu.VMEM((per_w,), jnp.int32),
            pltpu.VMEM((_CHUNK, d), jnp.float32),
            pltpu.SemaphoreType.DMA,
        ],
    )
    def gather_k(tbl_hbm, idx_hbm, out_hbm, idx_v, rows_v, sem):
        wid = lax.axis_index("s") * _NUM_CORES + lax.axis_index("c")
        base = wid * per_w
        pltpu.sync_copy(idx_hbm.at[pl.ds(base, per_w)], idx_v)

        def chunk(j, carry):
            off = j * _CHUNK
            pltpu.async_copy(
                tbl_hbm.at[idx_v.at[pl.ds(off, _CHUNK)]], rows_v, sem).wait()
            pltpu.sync_copy(rows_v, out_hbm.at[pl.ds(base + off, _CHUNK)])
            return carry

        lax.fori_loop(0, per_w // _CHUNK, chunk, 0)

    return gather_k(norm_table, idx_flat)


def _lstm_body(emb_ref, wcat_ref, bias_ref, wcls_ref, bcls_ref, out_ref,
               h_scr, c_scr):
    t = pl.program_id(0)
    nt = pl.num_programs(0)
    h4 = bias_ref.shape[1]
    hdim = h4 // 4

    @pl.when(t == 0)
    def _():
        h_scr[...] = jnp.zeros_like(h_scr)
        c_scr[...] = jnp.zeros_like(c_scr)

    xt = emb_ref[0]
    h = h_scr[...]
    cat = jnp.concatenate([xt, h], axis=1)
    gates = jnp.dot(cat, wcat_ref[...],
                    preferred_element_type=jnp.float32) + bias_ref[...]
    i = jax.nn.sigmoid(gates[:, 0 * hdim:1 * hdim])
    f = jax.nn.sigmoid(gates[:, 1 * hdim:2 * hdim])
    g = jnp.tanh(gates[:, 2 * hdim:3 * hdim])
    o = jax.nn.sigmoid(gates[:, 3 * hdim:4 * hdim])
    c = f * c_scr[...] + i * g
    hn = o * jnp.tanh(c)
    h_scr[...] = hn
    c_scr[...] = c

    @pl.when(t == nt - 1)
    def _():
        out_ref[...] = jnp.dot(hn, wcls_ref[...],
                               preferred_element_type=jnp.float32) + bcls_ref[...]


_N_CHUNKS = 2


def kernel(x, table, gamma, beta, W_ih, W_hh, b_ih, b_hh, W_cls, b_cls):
    B, T = x.shape
    V, D = table.shape
    H = W_hh.shape[1]
    C = W_cls.shape[0]
    pad = _LANE - D

    tbl128 = jnp.pad(table, ((0, 0), (0, pad)))
    gamma128 = jnp.pad(gamma.reshape(1, D), ((0, 0), (0, pad)))
    beta128 = jnp.pad(beta.reshape(1, D), ((0, 0), (0, pad)))

    norm_table = pl.pallas_call(
        _make_ln_table_body(D),
        out_shape=jax.ShapeDtypeStruct((V, _LANE), jnp.float32),
    )(tbl128, gamma128, beta128)

    wcat = jnp.concatenate(
        [W_ih.T, jnp.zeros((pad, 4 * H), jnp.float32), W_hh.T], axis=0)
    bias = (b_ih + b_hh).reshape(1, 4 * H)
    wcls = W_cls.T
    bcls = b_cls.reshape(1, C)

    bc = B // _N_CHUNKS
    lstm = pl.pallas_call(
        _lstm_body,
        grid=(T,),
        in_specs=[
            pl.BlockSpec((1, bc, _LANE), lambda t: (t, 0, 0)),
            pl.BlockSpec((_LANE + H, 4 * H), lambda t: (0, 0)),
            pl.BlockSpec((1, 4 * H), lambda t: (0, 0)),
            pl.BlockSpec((H, C), lambda t: (0, 0)),
            pl.BlockSpec((1, C), lambda t: (0, 0)),
        ],
        out_specs=pl.BlockSpec((bc, C), lambda t: (0, 0)),
        out_shape=jax.ShapeDtypeStruct((bc, C), jnp.float32),
        scratch_shapes=[
            pltpu.VMEM((bc, H), jnp.float32),
            pltpu.VMEM((bc, H), jnp.float32),
        ],
    )

    outs = []
    for k in range(_N_CHUNKS):
        xk = lax.slice_in_dim(x, k * bc, (k + 1) * bc, axis=0)
        idx_k = xk.T.reshape(-1)
        emb_k = _sc_gather(norm_table, idx_k).reshape(T, bc, _LANE)
        outs.append(lstm(emb_k, wcat, bias, wcls, bcls))
    return jnp.concatenate(outs, axis=0)
